# baseline (speedup 1.0000x reference)
import functools

import jax
import jax.numpy as jnp
from jax import lax
from jax.experimental import pallas as pl
from jax.experimental.pallas import tpu as pltpu
from jax.experimental.pallas import tpu_sc as plsc

N = 10000
ACC_N = 10240
IN_DIM = 256
HID = 64
OUT = 16
D1 = HID + 16
D2 = OUT

NUM_CORES = 2
NUM_TILES = 16
NUM_WORKERS = NUM_CORES * NUM_TILES
CHUNK = 128
ROWS_PER_TILE = ACC_N // NUM_TILES


def _mm1_body(x_ref, wl_ref, wr_ref, b_ref, ya_ref, xr_ref):
    x = x_ref[...]
    xl = jnp.dot(x, wl_ref[...], preferred_element_type=jnp.float32,
                 precision=lax.Precision.HIGHEST)
    blk = xl.shape[0]
    ya_ref[...] = jnp.concatenate(
        [xl, jnp.ones((blk, D1 - HID), jnp.float32)], axis=1)
    xr_ref[...] = jnp.dot(x, wr_ref[...], preferred_element_type=jnp.float32,
                          precision=lax.Precision.HIGHEST) + b_ref[...]


def _mm1(x, wlT, wrT, b1l2d):
    blk = 2000
    return pl.pallas_call(
        _mm1_body,
        grid=(N // blk,),
        in_specs=[
            pl.BlockSpec((blk, IN_DIM), lambda i: (i, 0)),
            pl.BlockSpec((IN_DIM, HID), lambda i: (0, 0)),
            pl.BlockSpec((IN_DIM, HID), lambda i: (0, 0)),
            pl.BlockSpec((1, HID), lambda i: (0, 0)),
        ],
        out_specs=[
            pl.BlockSpec((blk, D1), lambda i: (i, 0)),
            pl.BlockSpec((blk, HID), lambda i: (i, 0)),
        ],
        out_shape=[
            jax.ShapeDtypeStruct((N, D1), jnp.float32),
            jax.ShapeDtypeStruct((N, HID), jnp.float32),
        ],
    )(x, wlT, wrT, b1l2d)


NBUF = 8

CPT = 80
ZROWS = 64


def _make_scatter(d, cpt):
    mesh = plsc.VectorSubcoreMesh(core_axis_name="c", subcore_axis_name="s",
                                  num_cores=1)
    nbuf = 4 if d > 32 else NBUF

    @functools.partial(
        pl.kernel,
        mesh=mesh,
        compiler_params=pltpu.CompilerParams(use_tc_tiling_on_sc=False),
        out_type=jax.ShapeDtypeStruct((ACC_N, d), jnp.float32),
        scratch_types=[
            pltpu.VMEM((cpt, CHUNK), jnp.int32),
            pltpu.VMEM((cpt, CHUNK), jnp.int32),
            pltpu.VMEM((nbuf, CHUNK, d), jnp.float32),
            pltpu.VMEM((ZROWS, d), jnp.float32),
            pltpu.VMEM_SHARED((ACC_N, d), jnp.float32),
            pltpu.SemaphoreType.DMA,
            pltpu.SemaphoreType.DMA,
            pltpu.SemaphoreType.DMA,
        ],
    )
    def scatter(table_hbm, src_hbm, dst_hbm, out_hbm,
                src_all, dst_all, rows, zbuf, acc_sh, gsem, ssem, zsem):
        sid = lax.axis_index("s")
        base_chunk = sid * cpt

        ip1 = pltpu.async_copy(src_hbm.at[pl.ds(base_chunk, cpt)],
                               src_all, gsem)
        ip2 = pltpu.async_copy(dst_hbm.at[pl.ds(base_chunk, cpt)],
                               dst_all, ssem)
        z16 = jnp.zeros((16,), jnp.float32)

        def zrow(r, carry):
            for j in range(d // 16):
                zbuf[r, pl.ds(j * 16, 16)] = z16
            return carry

        lax.fori_loop(0, ZROWS, zrow, 0)
        zcps = [
            pltpu.async_copy(
                zbuf,
                acc_sh.at[pl.ds(sid * ROWS_PER_TILE + t * ZROWS, ZROWS)],
                zsem)
            for t in range(ROWS_PER_TILE // ZROWS)
        ]
        for cp in zcps:
            cp.wait()
        ip1.wait()
        ip2.wait()
        plsc.subcore_barrier()

        def outer(g, carry):
            base = g * nbuf
            gcps = [
                pltpu.async_copy(table_hbm.at[src_all.at[base + b]],
                                 rows.at[b], gsem)
                for b in range(nbuf)
            ]
            scps = []
            for b in range(nbuf):
                gcps[b].wait()
                scps.append(
                    pltpu.async_copy(rows.at[b],
                                     acc_sh.at[dst_all.at[base + b]],
                                     ssem, add=True))
            for cp in scps:
                cp.wait()
            return carry

        lax.fori_loop(0, cpt // nbuf, outer, 0)
        plsc.subcore_barrier()

        pltpu.sync_copy(
            acc_sh.at[pl.ds(sid * ROWS_PER_TILE, ROWS_PER_TILE)],
            out_hbm.at[pl.ds(sid * ROWS_PER_TILE, ROWS_PER_TILE)])

    return scatter


def _mid_body(acc_ref, ya_ref, xr_ref, g_ref, b_ref, w2_ref, b2_ref,
              z1_ref, zr_ref, rcnt_ref):
    s = acc_ref[...][:N]
    cnt = s[:, HID:HID + 1] + 1.0
    rcnt = 1.0 / cnt
    y1 = ya_ref[...][:, :HID]
    h = (s[:, :HID] + y1) * rcnt + xr_ref[...]
    mean = jnp.mean(h, axis=0, keepdims=True)
    var = jnp.mean((h - mean) ** 2, axis=0, keepdims=True)
    hn = (h - mean) * lax.rsqrt(var + 1e-5) * g_ref[...] + b_ref[...]
    hn = jnp.maximum(hn, 0.0)
    z = jnp.dot(hn, w2_ref[...], preferred_element_type=jnp.float32,
                precision=lax.Precision.HIGHEST)
    z1_ref[...] = z[:, :OUT]
    zr_ref[...] = z[:, OUT:] + b2_ref[...]
    rcnt_ref[...] = jnp.broadcast_to(rcnt, (N, OUT))


def _mid(acc, ya, xr, gamma2d, beta2d, w2cat, b2l2d):
    return pl.pallas_call(
        _mid_body,
        out_shape=[
            jax.ShapeDtypeStruct((N, OUT), jnp.float32),
            jax.ShapeDtypeStruct((N, OUT), jnp.float32),
            jax.ShapeDtypeStruct((N, OUT), jnp.float32),
        ],
    )(acc, ya, xr, gamma2d, beta2d, w2cat, b2l2d)


def _fin_body(acc2_ref, z1_ref, rcnt_ref, zr_ref, out_ref):
    s = acc2_ref[...][:N]
    out_ref[...] = (s + z1_ref[...]) * rcnt_ref[...] + zr_ref[...]


def _fin(acc2, z1, rcnt, zr):
    return pl.pallas_call(
        _fin_body,
        out_shape=jax.ShapeDtypeStruct((N, OUT), jnp.float32),
    )(acc2, z1, rcnt, zr)


def kernel(x, edge_index, W1l, b1l, W1r, gamma, beta, W2l, b2l, W2r):
    e = edge_index.shape[1]
    apad = NUM_TILES * CPT * CHUNK
    assert apad >= e and CPT % 8 == 0

    src_p = jnp.concatenate(
        [edge_index[0], jnp.zeros((apad - e,), jnp.int32)]
    ).reshape(apad // CHUNK, CHUNK)
    dst_p = jnp.concatenate(
        [edge_index[1], jnp.full((apad - e,), N, jnp.int32)]
    ).reshape(apad // CHUNK, CHUNK)
    ya, xr = _mm1(x, W1l.T, W1r.T, b1l[None, :])
    acc = _make_scatter(D1, CPT)(ya, src_p, dst_p)
    w2cat = jnp.concatenate([W2l.T, W2r.T], axis=1)
    z1, zr, rcnt = _mid(acc, ya, xr, gamma[None, :], beta[None, :],
                        w2cat, b2l[None, :])
    acc2 = _make_scatter(D2, CPT)(z1, src_p, dst_p)
    return _fin(acc2, z1, rcnt, zr)


# speedup vs baseline: 1.3129x; 1.3129x over previous
import functools

import jax
import jax.numpy as jnp
from jax import lax
from jax.experimental import pallas as pl
from jax.experimental.pallas import tpu as pltpu
from jax.experimental.pallas import tpu_sc as plsc

N = 10000
ACC_N = 10240
IN_DIM = 256
HID = 64
OUT = 16
D1 = HID + 16
D2 = OUT

NUM_CORES = 2
NUM_TILES = 16
NUM_WORKERS = NUM_CORES * NUM_TILES
CHUNK = 128
ROWS_PER_TILE = ACC_N // NUM_TILES


def _mm1_body(x_ref, wl_ref, wr_ref, b_ref, ya_ref, xr_ref):
    x = x_ref[...]
    xl = jnp.dot(x, wl_ref[...], preferred_element_type=jnp.float32,
                 precision=lax.Precision.HIGHEST)
    blk = xl.shape[0]
    ya_ref[...] = jnp.concatenate(
        [xl, jnp.ones((blk, D1 - HID), jnp.float32)], axis=1)
    xr_ref[...] = jnp.dot(x, wr_ref[...], preferred_element_type=jnp.float32,
                          precision=lax.Precision.HIGHEST) + b_ref[...]


def _mm1(x, wlT, wrT, b1l2d):
    blk = 2000
    return pl.pallas_call(
        _mm1_body,
        grid=(N // blk,),
        in_specs=[
            pl.BlockSpec((blk, IN_DIM), lambda i: (i, 0)),
            pl.BlockSpec((IN_DIM, HID), lambda i: (0, 0)),
            pl.BlockSpec((IN_DIM, HID), lambda i: (0, 0)),
            pl.BlockSpec((1, HID), lambda i: (0, 0)),
        ],
        out_specs=[
            pl.BlockSpec((blk, D1), lambda i: (i, 0)),
            pl.BlockSpec((blk, HID), lambda i: (i, 0)),
        ],
        out_shape=[
            jax.ShapeDtypeStruct((N, D1), jnp.float32),
            jax.ShapeDtypeStruct((N, HID), jnp.float32),
        ],
    )(x, wlT, wrT, b1l2d)


NBUF = 8

CPT0 = 72
CPT1 = 8
ZROWS = 64


def _make_scatter(d, cpt0, cpt1):
    mesh = plsc.VectorSubcoreMesh(core_axis_name="c", subcore_axis_name="s")
    nbuf = 4 if d > 32 else NBUF

    @functools.partial(
        pl.kernel,
        mesh=mesh,
        compiler_params=pltpu.CompilerParams(use_tc_tiling_on_sc=False),
        out_type=jax.ShapeDtypeStruct((NUM_CORES * ACC_N, d), jnp.float32),
        scratch_types=[
            pltpu.VMEM((cpt0, CHUNK), jnp.int32),
            pltpu.VMEM((cpt0, CHUNK), jnp.int32),
            pltpu.VMEM((nbuf, CHUNK, d), jnp.float32),
            pltpu.VMEM((ZROWS, d), jnp.float32),
            pltpu.VMEM_SHARED((ACC_N, d), jnp.float32),
            pltpu.SemaphoreType.DMA,
            pltpu.SemaphoreType.DMA,
            pltpu.SemaphoreType.DMA,
        ],
    )
    def scatter(table_hbm, src_hbm, dst_hbm, out_hbm,
                src_all, dst_all, rows, zbuf, acc_sh, gsem, ssem, zsem):
        cid = lax.axis_index("c")
        sid = lax.axis_index("s")
        cpt_c = jnp.where(cid == 0, cpt0, cpt1)
        base_chunk = cid * NUM_TILES * cpt0 + sid * cpt_c

        ip1 = pltpu.async_copy(src_hbm.at[pl.ds(base_chunk, cpt0)],
                               src_all, gsem)
        ip2 = pltpu.async_copy(dst_hbm.at[pl.ds(base_chunk, cpt0)],
                               dst_all, ssem)
        z16 = jnp.zeros((16,), jnp.float32)

        def zrow(r, carry):
            for j in range(d // 16):
                zbuf[r, pl.ds(j * 16, 16)] = z16
            return carry

        lax.fori_loop(0, ZROWS, zrow, 0)
        zcps = [
            pltpu.async_copy(
                zbuf,
                acc_sh.at[pl.ds(sid * ROWS_PER_TILE + t * ZROWS, ZROWS)],
                zsem)
            for t in range(ROWS_PER_TILE // ZROWS)
        ]
        for cp in zcps:
            cp.wait()
        ip1.wait()
        ip2.wait()
        plsc.subcore_barrier()

        def outer(g, carry):
            base = g * nbuf
            gcps = [
                pltpu.async_copy(table_hbm.at[src_all.at[base + b]],
                                 rows.at[b], gsem)
                for b in range(nbuf)
            ]
            scps = []
            for b in range(nbuf):
                gcps[b].wait()
                scps.append(
                    pltpu.async_copy(rows.at[b],
                                     acc_sh.at[dst_all.at[base + b]],
                                     ssem, add=True))
            for cp in scps:
                cp.wait()
            return carry

        lax.fori_loop(0, cpt_c // nbuf, outer, 0)
        plsc.subcore_barrier()

        pltpu.sync_copy(
            acc_sh.at[pl.ds(sid * ROWS_PER_TILE, ROWS_PER_TILE)],
            out_hbm.at[pl.ds(cid * ACC_N + sid * ROWS_PER_TILE,
                             ROWS_PER_TILE)])

    return scatter


def _mid_body(acc_ref, ya_ref, xr_ref, g_ref, b_ref, w2_ref, b2_ref,
              z1_ref, zr_ref, rcnt_ref):
    a = acc_ref[...]
    s = a[:N] + a[ACC_N:ACC_N + N]
    cnt = s[:, HID:HID + 1] + 1.0
    rcnt = 1.0 / cnt
    y1 = ya_ref[...][:, :HID]
    h = (s[:, :HID] + y1) * rcnt + xr_ref[...]
    mean = jnp.mean(h, axis=0, keepdims=True)
    var = jnp.mean((h - mean) ** 2, axis=0, keepdims=True)
    hn = (h - mean) * lax.rsqrt(var + 1e-5) * g_ref[...] + b_ref[...]
    hn = jnp.maximum(hn, 0.0)
    z = jnp.dot(hn, w2_ref[...], preferred_element_type=jnp.float32,
                precision=lax.Precision.HIGHEST)
    z1_ref[...] = z[:, :OUT]
    zr_ref[...] = z[:, OUT:] + b2_ref[...]
    rcnt_ref[...] = jnp.broadcast_to(rcnt, (N, OUT))


def _mid(acc, ya, xr, gamma2d, beta2d, w2cat, b2l2d):
    return pl.pallas_call(
        _mid_body,
        out_shape=[
            jax.ShapeDtypeStruct((N, OUT), jnp.float32),
            jax.ShapeDtypeStruct((N, OUT), jnp.float32),
            jax.ShapeDtypeStruct((N, OUT), jnp.float32),
        ],
    )(acc, ya, xr, gamma2d, beta2d, w2cat, b2l2d)


def _fin_body(acc2_ref, z1_ref, rcnt_ref, zr_ref, out_ref):
    a = acc2_ref[...]
    s = a[:N] + a[ACC_N:ACC_N + N]
    out_ref[...] = (s + z1_ref[...]) * rcnt_ref[...] + zr_ref[...]


def _fin(acc2, z1, rcnt, zr):
    return pl.pallas_call(
        _fin_body,
        out_shape=jax.ShapeDtypeStruct((N, OUT), jnp.float32),
    )(acc2, z1, rcnt, zr)


def kernel(x, edge_index, W1l, b1l, W1r, gamma, beta, W2l, b2l, W2r):
    e = edge_index.shape[1]
    epad = NUM_TILES * (CPT0 + CPT1) * CHUNK
    assert epad >= e and CPT0 % 8 == 0 and CPT1 % 8 == 0
    apad = epad + CPT0 * CHUNK

    src_p = jnp.concatenate(
        [edge_index[0], jnp.zeros((apad - e,), jnp.int32)]
    ).reshape(apad // CHUNK, CHUNK)
    dst_p = jnp.concatenate(
        [edge_index[1], jnp.full((apad - e,), N, jnp.int32)]
    ).reshape(apad // CHUNK, CHUNK)
    ya, xr = _mm1(x, W1l.T, W1r.T, b1l[None, :])
    acc = _make_scatter(D1, CPT0, CPT1)(ya, src_p, dst_p)
    w2cat = jnp.concatenate([W2l.T, W2r.T], axis=1)
    z1, zr, rcnt = _mid(acc, ya, xr, gamma[None, :], beta[None, :],
                        w2cat, b2l[None, :])
    acc2 = _make_scatter(D2, CPT0, CPT1)(z1, src_p, dst_p)
    return _fin(acc2, z1, rcnt, zr)
